# baseline (speedup 1.0000x reference)
import jax
import jax.numpy as jnp
from jax.experimental import pallas as pl
from jax.experimental.pallas import tpu as pltpu


def _gather_body(idx_ref, w_ref, o_ref):
    bm, S = o_ref.shape[0], o_ref.shape[1]
    for b in range(bm):
        for s in range(S):
            t = idx_ref[b, s]
            chunk = w_ref[t >> 3]
            rolled = pltpu.roll(chunk, (s - t) & 7, axis=0)
            o_ref[b, s] = rolled[s % 8]


def kernel(indices, weight):
    B, S = indices.shape
    V, D = weight.shape
    idx = indices.astype(jnp.int32)

    bm = 8
    n_steps = B // bm

    w3 = weight.reshape(V // 8, 8, D)

    table_bytes = V * D * jnp.dtype(weight.dtype).itemsize
    out_block_bytes = bm * S * D * jnp.dtype(weight.dtype).itemsize
    vmem_limit = int(min(table_bytes + 4 * out_block_bytes + (4 << 20),
                         100 * 1024 * 1024))

    return pl.pallas_call(
        _gather_body,
        out_shape=jax.ShapeDtypeStruct((B, S, D), weight.dtype),
        grid=(n_steps,),
        in_specs=[
            pl.BlockSpec((bm, S), lambda i: (i, 0), memory_space=pltpu.SMEM),
            pl.BlockSpec((V // 8, 8, D), lambda i: (0, 0, 0)),
        ],
        out_specs=pl.BlockSpec((bm, S, D), lambda i: (i, 0, 0)),
        compiler_params=pltpu.CompilerParams(
            dimension_semantics=("parallel",),
            vmem_limit_bytes=vmem_limit,
        ),
    )(idx, w3)


# speedup vs baseline: 1.4422x; 1.4422x over previous
import jax
import jax.numpy as jnp
from jax.experimental import pallas as pl
from jax.experimental.pallas import tpu as pltpu


def _gather_body(idx_ref, w_ref, o_ref):
    bm, S = o_ref.shape[0], o_ref.shape[1]
    for b in range(bm):
        for s in range(S):
            o_ref[b, s] = w_ref[idx_ref[b * S + s], 0]


def kernel(indices, weight):
    B, S = indices.shape
    V, D = weight.shape
    flat_idx = indices.reshape(-1).astype(jnp.int32)

    bm = max(1, 2048 // S)
    while B % bm:
        bm -= 1
    m = bm * S
    n_steps = B // bm

    w3 = weight.reshape(V, 1, D)

    table_bytes = V * D * jnp.dtype(weight.dtype).itemsize
    out_block_bytes = m * D * jnp.dtype(weight.dtype).itemsize
    vmem_limit = int(min(table_bytes + 4 * out_block_bytes + (4 << 20),
                         100 * 1024 * 1024))

    return pl.pallas_call(
        _gather_body,
        out_shape=jax.ShapeDtypeStruct((B, S, D), weight.dtype),
        grid=(2, n_steps // 2),
        in_specs=[
            pl.BlockSpec((m,), lambda c, i: (c * (n_steps // 2) + i,),
                         memory_space=pltpu.SMEM),
            pl.BlockSpec((V, 1, D), lambda c, i: (0, 0, 0)),
        ],
        out_specs=pl.BlockSpec((bm, S, D),
                               lambda c, i: (c * (n_steps // 2) + i, 0, 0)),
        compiler_params=pltpu.CompilerParams(
            dimension_semantics=("parallel", "arbitrary"),
            vmem_limit_bytes=vmem_limit,
        ),
    )(flat_idx, w3)


# speedup vs baseline: 1.6268x; 1.1280x over previous
import jax
import jax.numpy as jnp
from jax.experimental import pallas as pl
from jax.experimental.pallas import tpu as pltpu


def _gather_body(idx_ref, w_ref, o_ref, tbl_ref):
    bm, S = o_ref.shape[0], o_ref.shape[1]
    V = tbl_ref.shape[0]

    @pl.when(pl.program_id(0) == 0)
    def _relayout():
        tbl_ref[...] = w_ref[...].reshape(V, 1, -1)

    for b in range(bm):
        for s in range(S):
            o_ref[b, s] = tbl_ref[idx_ref[b, s], 0]


def kernel(indices, weight):
    B, S = indices.shape
    V, D = weight.shape
    idx = indices.astype(jnp.int32)

    bm = 8
    n_steps = B // bm

    w3 = weight.reshape(V // 8, 8, D)

    table_bytes = V * D * jnp.dtype(weight.dtype).itemsize
    out_block_bytes = bm * S * D * jnp.dtype(weight.dtype).itemsize
    vmem_limit = int(min(2 * table_bytes + 2 * out_block_bytes + (4 << 20),
                         128 * 1024 * 1024))

    return pl.pallas_call(
        _gather_body,
        out_shape=jax.ShapeDtypeStruct((B, S, D), weight.dtype),
        grid=(n_steps,),
        in_specs=[
            pl.BlockSpec((bm, S), lambda i: (i, 0), memory_space=pltpu.SMEM),
            pl.BlockSpec((V // 8, 8, D), lambda i: (0, 0, 0)),
        ],
        out_specs=pl.BlockSpec((bm, S, D), lambda i: (i, 0, 0)),
        scratch_shapes=[pltpu.VMEM((V, 1, D), jnp.float32)],
        compiler_params=pltpu.CompilerParams(
            dimension_semantics=("arbitrary",),
            vmem_limit_bytes=vmem_limit,
        ),
    )(idx, w3)
